# baseline (speedup 1.0000x reference)
import functools

import jax
import jax.numpy as jnp
from jax import lax
from jax.experimental import pallas as pl
from jax.experimental.pallas import tpu as pltpu
from jax.experimental.pallas import tpu_sc as plsc

DIM = 64
B = 1024
L = 200
LB = L // 8
BC = B // 128
UNITS = LB * BC

_info = plsc.get_sparse_core_info()
NC, NS = _info.num_cores, _info.num_subcores
NW = NC * NS
KMAX = (UNITS + NW - 1) // NW


@functools.partial(
    pl.kernel,
    out_type=jax.ShapeDtypeStruct((L, DIM, B), jnp.float32),
    mesh=plsc.VectorSubcoreMesh(core_axis_name="c", subcore_axis_name="s"),
    compiler_params=pltpu.CompilerParams(needs_layout_passes=False),
    scratch_types=[
        pltpu.VMEM((8, 128), jnp.int32),
        pltpu.VMEM((2, 128), jnp.int32),
        pltpu.VMEM((2, 128), jnp.int32),
        pltpu.VMEM((2, 128, 128), jnp.float32),
        pltpu.VMEM((2, DIM, 128), jnp.float32),
        pltpu.SemaphoreType.DMA,
        pltpu.SemaphoreType.DMA,
    ],
)
def _gather_kernel(table_hbm, idx_hbm, out_hbm, idx_v, gidx_v, pb_v, gbuf,
                   oblk, gsem, wsem):
    wid = lax.axis_index("s") * NC + lax.axis_index("c")

    def compute_idx(s, gslot):
        for jg in range(8):
            v16 = idx_v[s, pl.ds(jg * 16, 16)]
            gidx_v[gslot, pl.ds(jg * 16, 16)] = v16 >> 1
            pb_v[gslot, pl.ds(jg * 16, 16)] = (v16 & 1) << 6

    def fire_gather(gslot):
        pltpu.async_copy(table_hbm.at[gidx_v.at[gslot]], gbuf.at[gslot], gsem)

    def extract(slot):
        for jg in range(8):
            rows16 = lax.iota(jnp.int32, 16) + 16 * jg
            pb16 = pb_v[slot, pl.ds(jg * 16, 16)]

            @pl.loop(0, DIM, unroll=8)
            def _(d):
                val = plsc.load_gather(gbuf.at[slot], [rows16, pb16 + d])
                oblk[slot, d, pl.ds(jg * 16, 16)] = val

    def drain_write():
        pltpu.make_async_copy(
            oblk.at[0], out_hbm.at[0, :, pl.ds(0, 128)], wsem
        ).wait()

    @pl.loop(0, KMAX)
    def _(k):
        u = wid + NW * k

        @pl.when(u < UNITS)
        def _():
            lb = u // BC
            bc = lax.rem(u, BC)
            pltpu.sync_copy(
                idx_hbm.at[pl.ds(lb * 8, 8), pl.ds(bc * 128, 128)], idx_v
            )
            compute_idx(0, 0)
            fire_gather(0)

            @pl.loop(0, 8)
            def _(s):
                slot = lax.rem(s, 2)
                nxt = 1 - slot

                @pl.when(s < 7)
                def _():
                    compute_idx(s + 1, nxt)
                    fire_gather(nxt)

                pltpu.make_async_copy(
                    table_hbm.at[gidx_v.at[slot]], gbuf.at[slot], gsem
                ).wait()

                @pl.when(s >= 2)
                def _():
                    drain_write()

                extract(slot)
                pltpu.async_copy(
                    oblk.at[slot],
                    out_hbm.at[lb * 8 + s, :, pl.ds(bc * 128, 128)],
                    wsem,
                )

            drain_write()
            drain_write()


def kernel(mode, word_inputs, word_seq_lengths, char_inputs, char_seq_lengths,
           char_seq_recover, word_embedding_weight):
    table_pairs = word_embedding_weight.astype(jnp.float32).reshape(
        500000, 128)
    idx_t = word_inputs.astype(jnp.int32).T
    out_t = _gather_kernel(table_pairs, idx_t)
    return jnp.transpose(out_t, (2, 0, 1))


# speedup vs baseline: 1.2658x; 1.2658x over previous
import functools

import jax
import jax.numpy as jnp
from jax import lax
from jax.experimental import pallas as pl
from jax.experimental.pallas import tpu as pltpu
from jax.experimental.pallas import tpu_sc as plsc

DIM = 64
B = 1024
L = 200

_info = plsc.get_sparse_core_info()
NC, NS = _info.num_cores, _info.num_subcores
NW = NC * NS
ROWS_W = B // NW
NBUF = 4


@functools.partial(
    pl.kernel,
    out_type=jax.ShapeDtypeStruct((B, L, DIM), jnp.float32),
    mesh=plsc.VectorSubcoreMesh(core_axis_name="c", subcore_axis_name="s"),
    compiler_params=pltpu.CompilerParams(use_tc_tiling_on_sc=False),
    scratch_types=[
        pltpu.VMEM((ROWS_W * L,), jnp.int32),
        pltpu.VMEM((NBUF, L, DIM), jnp.float32),
        pltpu.SemaphoreType.DMA,
        pltpu.SemaphoreType.DMA,
        pltpu.SemaphoreType.DMA,
    ],
)
def _gather_kernel(table_hbm, idx_hbm, out_hbm, idx_v, rows_v, isem, gsem,
                   wsem):
    wid = lax.axis_index("s") * NC + lax.axis_index("c")
    rbase = wid * ROWS_W

    for m in range(ROWS_W):
        pltpu.async_copy(idx_hbm.at[rbase + m], idx_v.at[pl.ds(m * L, L)],
                         isem)
    for m in range(ROWS_W):
        pltpu.make_async_copy(idx_hbm.at[rbase], idx_v.at[pl.ds(0, L)],
                              isem).wait()

    for b in range(NBUF):
        pltpu.async_copy(
            table_hbm.at[idx_v.at[pl.ds(b * L, L)]], rows_v.at[b], gsem
        )

    @pl.loop(0, ROWS_W)
    def _(j):
        slot = lax.rem(j, NBUF)
        pltpu.make_async_copy(
            table_hbm.at[idx_v.at[pl.ds(0, L)]], rows_v.at[slot], gsem
        ).wait()
        write = pltpu.async_copy(rows_v.at[slot], out_hbm.at[rbase + j], wsem)

        @pl.when(j + NBUF < ROWS_W)
        def _():
            write.wait()
            pltpu.async_copy(
                table_hbm.at[idx_v.at[pl.ds((j + NBUF) * L, L)]],
                rows_v.at[slot],
                gsem,
            )

    for b in range(NBUF):
        pltpu.make_async_copy(rows_v.at[b], out_hbm.at[rbase], wsem).wait()


def kernel(mode, word_inputs, word_seq_lengths, char_inputs, char_seq_lengths,
           char_seq_recover, word_embedding_weight):
    idx = word_inputs.astype(jnp.int32)
    return _gather_kernel(word_embedding_weight, idx)
